# Initial kernel scaffold; baseline (speedup 1.0000x reference)
import functools

import jax
import jax.numpy as jnp
import numpy as np
from jax import lax
from jax.experimental import pallas as pl
from jax.experimental.pallas import tpu as pltpu
from jax.experimental.pallas import tpu_sc as plsc

_B = 2
_P = 4
_C = 3
_H = 128
_D = 32
_N = _D * _D * _D
_NNZ = 16 * _N
_FEAT = [132, 384, 192, 192, 32]
_FC = 16

_LV = {}
for _d in (32, 16, 8, 4):
    _dp = _d + 2
    _sp = _dp ** 3
    _off = _dp * _dp + _dp + 1
    _LV[_d] = (_dp, _sp, _off, _sp + 2 * _off)


def _mask_for(d):
    dp, sp, off, rtot = _LV[d]
    m3 = np.zeros((dp, dp, dp), np.float32)
    m3[1:-1, 1:-1, 1:-1] = 1.0
    m = np.zeros((1, rtot), np.float32)
    m[0, off:off + sp] = m3.reshape(-1)
    return jnp.asarray(m)


_MASKS = {d: _mask_for(d) for d in (32, 16, 8, 4)}


def _embed(x5):
    d = x5.shape[-1]
    dp, sp, off, rtot = _LV[d]
    nb = x5.ndim - 3
    pads = [(0, 0)] * nb + [(1, 1)] * 3
    xp = jnp.pad(x5, pads).reshape(x5.shape[:nb] + (sp,))
    return jnp.pad(xp, [(0, 0)] * nb + [(off, off)])


def _interior(xf, d):
    dp, sp, off, rtot = _LV[d]
    x3 = xf[..., off:off + sp].reshape(xf.shape[:-1] + (dp, dp, dp))
    return x3[..., 1:-1, 1:-1, 1:-1]


def _pool_slices(xf, d):
    x5 = _interior(xf, d)
    outs = []
    for a in (0, 1):
        for b in (0, 1):
            for c in (0, 1):
                outs.append(_embed(x5[..., a::2, b::2, c::2]))
    return jnp.stack(outs, axis=0)


def _conv_w(w):
    return jnp.transpose(w, (2, 3, 4, 0, 1)).reshape(27, w.shape[0], w.shape[1])


def _tconv_w(tw):
    twf = tw[:, :, ::-1, ::-1, ::-1]
    return jnp.transpose(twf, (2, 3, 4, 0, 1)).reshape(8, tw.shape[0], tw.shape[1])


def _col(v):
    return v.reshape(-1, 1)


def _row(v):
    return v.reshape(1, -1)


def _prep_body(lpat, li, v0, v1, v2, lts_out, inv_out):
    lp = jax.nn.sigmoid(lpat[...])
    n = jnp.sqrt(jnp.sum(lp * lp))
    lts_out[...] = jnp.dot(lp / n, li[...], preferred_element_type=jnp.float32)
    mx = [jnp.max(v[...]).reshape(1, 1) for v in (v0, v1, v2)]
    inv_out[...] = 1.0 / jnp.concatenate(mx, axis=0)


def _prep_call(lightpattern, light_info, vals):
    li = light_info.reshape(_H, _D * _D)
    v3 = [v.reshape(-1, 128) for v in vals]
    return pl.pallas_call(
        _prep_body,
        out_shape=[
            jax.ShapeDtypeStruct((_P, _D * _D), jnp.float32),
            jax.ShapeDtypeStruct((3, 1), jnp.float32),
        ],
    )(lightpattern, li, *v3)


def _sc_spmm_body(W, t0, t1, t2, g0, g1, g2, s0, s1, s2, v0, v1, v2, z,
                  o0, o1, o2, sh0, sh1, sh2, gi_v, si_v, va_v, gb, pb, sem):
    cid = lax.axis_index("c")
    sid = lax.axis_index("s")
    wid = cid * 16 + sid
    tables = (t0, t1, t2)
    gidx = (g0, g1, g2)
    sidx = (s0, s1, s2)
    vals = (v0, v1, v2)
    outs = (o0, o1, o2)
    shared = (sh0, sh1, sh2)
    stripe = _N // 16

    for m in range(3):
        pltpu.sync_copy(z, shared[m].at[pl.ds(sid * stripe, stripe)])
    plsc.subcore_barrier()

    iota = lax.broadcasted_iota(jnp.int32, (16,), 0)
    lw = {4: 2, 2: 1}[W]
    nt = (128 * W) // 16
    jrows = (_NNZ // 32) // 128

    for m in range(3):
        row0 = wid * jrows
        pltpu.sync_copy(gidx[m].at[pl.ds(row0, jrows)], gi_v)
        pltpu.sync_copy(sidx[m].at[pl.ds(row0, jrows)], si_v)
        pltpu.sync_copy(vals[m].at[pl.ds(row0, jrows)], va_v)

        def body(j, _, m=m):
            pltpu.async_copy(tables[m].at[gi_v.at[j]], gb, sem).wait()
            jv = jnp.full((16,), j, jnp.int32)
            for t in range(nt):
                p = iota + t * 16
                r = lax.shift_right_logical(p, lw)
                c = lax.bitwise_and(p, W - 1)
                gg = plsc.load_gather(gb, [r, c])
                vv = plsc.load_gather(va_v, [jv, r])
                plsc.store_scatter(pb, [r, c], gg * vv)
            pltpu.sync_copy(pb, shared[m].at[si_v.at[j]], add=True)
            return 0

        lax.fori_loop(0, jrows, body, 0)

    plsc.subcore_barrier()
    for m in range(3):
        pltpu.sync_copy(shared[m].at[pl.ds(sid * stripe, stripe)],
                        outs[m].at[pl.ds(cid * _N + sid * stripe, stripe)])


def _sc_spmm_call(tables, gidx, sidx, vals, W):
    mesh = plsc.VectorSubcoreMesh(core_axis_name="c", subcore_axis_name="s")
    nr = _NNZ // 128
    gi = [g.reshape(nr, 128) for g in gidx]
    si = [s.reshape(nr, 128) for s in sidx]
    va = [v.reshape(nr, 128) for v in vals]
    z = jnp.zeros((_N // 16, W), jnp.float32)
    jrows = (_NNZ // 32) // 128
    fn = pl.kernel(
        functools.partial(_sc_spmm_body, W),
        out_type=[jax.ShapeDtypeStruct((2 * _N, W), jnp.float32)] * 3,
        mesh=mesh,
        scratch_types=[
            pltpu.VMEM_SHARED((_N, W), jnp.float32),
            pltpu.VMEM_SHARED((_N, W), jnp.float32),
            pltpu.VMEM_SHARED((_N, W), jnp.float32),
            pltpu.VMEM((jrows, 128), jnp.int32),
            pltpu.VMEM((jrows, 128), jnp.int32),
            pltpu.VMEM((jrows, 128), jnp.float32),
            pltpu.VMEM((128, W), jnp.float32),
            pltpu.VMEM((128, W), jnp.float32),
            pltpu.SemaphoreType.DMA,
        ],
    )
    return fn(*tables, *gi, *si, *va, z)


def _mlp_body(xp, c0, c1, invr, w0a, w0b, b0, g0, e0, w1, b1, g1, e1,
              w2, b2, g2, e2, w3, b3, g3, e3, out):
    def bn_lrelu(h, g, e):
        m = jnp.mean(h, axis=0, keepdims=True)
        ym = h - m
        v = jnp.mean(ym * ym, axis=0, keepdims=True)
        t = g[...] * ym * lax.rsqrt(v + 1e-5) + e[...]
        return jnp.where(t >= 0, t, 0.01 * t)

    c = (c0[...] + c1[...]) * invr[...]
    h = (jnp.dot(xp[...], w0a[...], preferred_element_type=jnp.float32)
         + jnp.dot(c, w0b[...], preferred_element_type=jnp.float32) + b0[...])
    h = bn_lrelu(h, g0, e0)
    for w, b, g, e in ((w1, b1, g1, e1), (w2, b2, g2, e2), (w3, b3, g3, e3)):
        h = jnp.dot(h, w[...], preferred_element_type=jnp.float32) + b[...]
        h = bn_lrelu(h, g, e)
    out[...] = h


def _mlp_call(xpart, cexp0, cexp1, invrows, p):
    args = [xpart, cexp0, cexp1, invrows,
            p['mlp_w0'][:_P], p['mlp_w0'][_P:], _row(p['mlp_b0']),
            _row(p['mlp_g0']), _row(p['mlp_e0'])]
    for i in (1, 2, 3):
        args += [p['mlp_w%d' % i], _row(p['mlp_b%d' % i]),
                 _row(p['mlp_g%d' % i]), _row(p['mlp_e%d' % i])]
    rows = 3 * _B * _D * _D
    return pl.pallas_call(
        _mlp_body,
        out_shape=jax.ShapeDtypeStruct((rows, _FEAT[4]), jnp.float32),
    )(*args)


def _conv_acc(x_list, w_list, d):
    dp, sp, off, rtot = _LV[d]
    acc = None
    for x, w in zip(x_list, w_list):
        ki = 0
        for dd in (-1, 0, 1):
            for dh in (-1, 0, 1):
                for dw in (-1, 0, 1):
                    rel = off + dd * dp * dp + dh * dp + dw
                    xs = x[:, rel:rel + sp]
                    t = jnp.dot(w[ki], xs, preferred_element_type=jnp.float32)
                    acc = t if acc is None else acc + t
                    ki += 1
    return acc


def _colsum(y):
    return jnp.sum(y, axis=1, keepdims=True)


def _bn_relu(ys, g, e, mask, cnt):
    s = ys[0] if len(ys) == 1 else ys[0] + ys[1]
    m = _colsum(s) / cnt
    ym = [(y - m) * mask for y in ys]
    sq = ym[0] * ym[0] if len(ym) == 1 else ym[0] * ym[0] + ym[1] * ym[1]
    v = _colsum(sq) / cnt
    sc = g * lax.rsqrt(v + 1e-5)
    return [jnp.maximum(t * sc + e, 0.0) * mask for t in ym]


def _pad_lanes(y, d):
    dp, sp, off, rtot = _LV[d]
    z = jnp.zeros((y.shape[0], off), jnp.float32)
    return jnp.concatenate([z, y, z], axis=1)


def _dcb(xs_by_b, w1, b1, g1, e1, w2, b2, g2, e2, mask, d):
    cnt = _B * (d ** 3)
    ys = [_pad_lanes(_conv_acc(xs, w1, d) + b1[...], d) * mask
          for xs in xs_by_b]
    ys = _bn_relu(ys, g1[...], e1[...], mask, cnt)
    ys = [_pad_lanes(_conv_acc([y], [w2[...]], d) + b2[...], d) * mask
          for y in ys]
    return _bn_relu(ys, g2[...], e2[...], mask, cnt)


def _dc_args(p, nm):
    return [_conv_w(p[nm + '_w1']), _col(p[nm + '_b1']),
            _col(p[nm + '_g1']), _col(p[nm + '_e1']),
            _conv_w(p[nm + '_w2']), _col(p[nm + '_b2']),
            _col(p[nm + '_g2']), _col(p[nm + '_e2'])]


def _enc0_body(d, vp0, vp1, inv, w1, b1, g1, e1, w2, b2, g2, e2, mask, out):
    xs_by_b = [[(vp0[b] + vp1[b]) * inv[...]] for b in range(_B)]
    ys = _dcb(xs_by_b, [w1], b1, g1, e1, w2, b2, g2, e2, mask[...], d)
    for b in range(_B):
        out[b] = ys[b]


def _enc0_call(vp0, vp1, inv, p):
    d = _D
    rtot = _LV[d][3]
    return pl.pallas_call(
        functools.partial(_enc0_body, d),
        out_shape=jax.ShapeDtypeStruct((_B, _FC, rtot), jnp.float32),
    )(vp0, vp1, inv, *_dc_args(p, 'inc'), _MASKS[d])


def _enc_body(d, p8, w1, b1, g1, e1, w2, b2, g2, e2, mask, out):
    xs_by_b = []
    for b in range(_B):
        x = functools.reduce(jnp.maximum, [p8[k, b] for k in range(8)])
        xs_by_b.append([x])
    ys = _dcb(xs_by_b, [w1], b1, g1, e1, w2, b2, g2, e2, mask[...], d)
    for b in range(_B):
        out[b] = ys[b]


def _enc_call(p8, nm, cout, d, p):
    rtot = _LV[d][3]
    return pl.pallas_call(
        functools.partial(_enc_body, d),
        out_shape=jax.ShapeDtypeStruct((_B, cout, rtot), jnp.float32),
    )(p8, *_dc_args(p, nm), _MASKS[d])


def _tconv_body(xd, tw, tb, out):
    for k in range(8):
        for b in range(_B):
            out[k, b] = (jnp.dot(tw[k], xd[b],
                                 preferred_element_type=jnp.float32)
                         + tb[...])


def _tconv_call(xf, nm, d_in, p):
    xd = _interior(xf, d_in).reshape(_B, xf.shape[1], d_in ** 3)
    tw = _tconv_w(p[nm + '_tw'])
    cout = tw.shape[1]
    y8 = pl.pallas_call(
        _tconv_body,
        out_shape=jax.ShapeDtypeStruct((8, _B, cout, d_in ** 3), jnp.float32),
    )(xd, tw, _col(p[nm + '_tb']))
    n = d_in
    y = y8.reshape(2, 2, 2, _B, cout, n, n, n)
    y = jnp.transpose(y, (3, 4, 5, 0, 6, 1, 7, 2)).reshape(_B, cout, 2 * n,
                                                           2 * n, 2 * n)
    return _embed(y)


def _up_body(d, skip, up, w1s, w1u, b1, g1, e1, w2, b2, g2, e2, mask, out):
    xs_by_b = [[skip[b], up[b]] for b in range(_B)]
    ys = _dcb(xs_by_b, [w1s, w1u], b1, g1, e1, w2, b2, g2, e2, mask[...], d)
    for b in range(_B):
        out[b] = ys[b]


def _up_call(skip, up, nm, cout, d, p):
    rtot = _LV[d][3]
    a = _dc_args(p, nm)
    w1 = a[0]
    cs = skip.shape[1]
    args = [skip, up, w1[:, :, :cs], w1[:, :, cs:]] + a[1:] + [_MASKS[d]]
    return pl.pallas_call(
        functools.partial(_up_body, d),
        out_shape=jax.ShapeDtypeStruct((_B, cout, rtot), jnp.float32),
    )(*args)


def _u4_body(d, skip, up, w1s, w1u, b1, g1, e1, w2, b2, g2, e2, mask,
             ow, ob, out):
    xs_by_b = [[skip[b], up[b]] for b in range(_B)]
    ys = _dcb(xs_by_b, [w1s, w1u], b1, g1, e1, w2, b2, g2, e2, mask[...], d)
    for b in range(_B):
        out[b] = (jnp.dot(ow[...], ys[b], preferred_element_type=jnp.float32)
                  + ob[...])


def _u4_call(skip, up, p):
    d = _D
    rtot = _LV[d][3]
    a = _dc_args(p, 'u4')
    w1 = a[0]
    cs = skip.shape[1]
    ow = p['out_w'].reshape(1, _FC)
    ob = p['out_b'].reshape(1, 1)
    args = [skip, up, w1[:, :, :cs], w1[:, :, cs:]] + a[1:] + [_MASKS[d], ow, ob]
    return pl.pallas_call(
        functools.partial(_u4_body, d),
        out_shape=jax.ShapeDtypeStruct((_B, 1, rtot), jnp.float32),
    )(*args)


def kernel(measurement, lightpattern, light_info, rows0, cols0, vals0,
           rows1, cols1, vals1, rows2, cols2, vals2, params):
    p = params
    rows = [rows0, rows1, rows2]
    cols = [cols0, cols1, cols2]
    vals = [vals0, vals1, vals2]

    lts, inv = _prep_call(lightpattern, light_info, vals)
    ltf = jnp.broadcast_to(lts.reshape(_P, _D, 1, _D), (_P, _D, _D, _D))
    ltf = jnp.transpose(ltf, (1, 2, 3, 0)).reshape(_N, _P)

    parts = _sc_spmm_call([ltf] * 3, cols, rows, vals, _P)

    m5 = measurement.reshape(_B, _P, _C, _D, _D)
    xpart = jnp.stack([jnp.transpose(m5[:, :, i], (0, 2, 3, 1))
                       for i in range(_C)], 0).reshape(-1, _P)
    cexp = []
    for half in range(2):
        c3 = jnp.stack([parts[i][half * _N:(half + 1) * _N].reshape(
            _D * _D, _D * _P) for i in range(_C)], 0)
        cexp.append(jnp.broadcast_to(c3[:, None], (_C, _B, _D * _D, _D * _P))
                    .reshape(-1, _D * _P))
    invrows = jnp.repeat(inv, _B * _D * _D, axis=0)
    y = _mlp_call(xpart, cexp[0], cexp[1], invrows, p)

    yb = jnp.transpose(y.reshape(_C, _B, _D, _D, _D), (0, 2, 3, 4, 1))
    yb = [yb[i].reshape(_N, _B) for i in range(_C)]
    tparts = _sc_spmm_call(yb, rows, cols, vals, _B)

    vp = []
    for half in range(2):
        v = jnp.stack([jnp.transpose(tparts[i][half * _N:(half + 1) * _N]
                                     .reshape(_D, _D, _D, _B), (3, 0, 1, 2))
                       for i in range(_C)], 1)
        vp.append(_embed(v))
    xx1 = _enc0_call(vp[0], vp[1], inv, p)
    xx2 = _enc_call(_pool_slices(xx1, 32), 'd1', _FC * 2, 16, p)
    xx3 = _enc_call(_pool_slices(xx2, 16), 'd2', _FC * 4, 8, p)
    xx4 = _enc_call(_pool_slices(xx3, 8), 'd3', _FC * 8, 4, p)
    u = _up_call(xx3, _tconv_call(xx4, 'u2', 4, p), 'u2', _FC * 4, 8, p)
    u = _up_call(xx2, _tconv_call(u, 'u3', 8, p), 'u3', _FC * 2, 16, p)
    fin = _u4_call(xx1, _tconv_call(u, 'u4', 16, p), p)
    return _interior(fin, _D).reshape(_B, 1, _D, _D, _D)


# speedup vs baseline: 3.9350x; 3.9350x over previous
import functools

import jax
import jax.numpy as jnp
import numpy as np
from jax import lax
from jax.experimental import pallas as pl
from jax.experimental.pallas import tpu as pltpu
from jax.experimental.pallas import tpu_sc as plsc

_B = 2
_P = 4
_C = 3
_H = 128
_D = 32
_N = _D * _D * _D
_NNZ = 16 * _N
_FEAT = [132, 384, 192, 192, 32]
_FC = 16

_LV = {}
for _d in (32, 16, 8, 4):
    _dp = _d + 2
    _sp = _dp ** 3
    _off = _dp * _dp + _dp + 1
    _LV[_d] = (_dp, _sp, _off, _sp + 2 * _off)


def _mask_for(d):
    dp, sp, off, rtot = _LV[d]
    m3 = np.zeros((dp, dp, dp), np.float32)
    m3[1:-1, 1:-1, 1:-1] = 1.0
    m = np.zeros((1, rtot), np.float32)
    m[0, off:off + sp] = m3.reshape(-1)
    return m


_MASKS = {d: _mask_for(d) for d in (32, 16, 8, 4)}


def _embed(x5):
    d = x5.shape[-1]
    dp, sp, off, rtot = _LV[d]
    nb = x5.ndim - 3
    pads = [(0, 0)] * nb + [(1, 1)] * 3
    xp = jnp.pad(x5, pads).reshape(x5.shape[:nb] + (sp,))
    return jnp.pad(xp, [(0, 0)] * nb + [(off, off)])


def _interior(xf, d):
    dp, sp, off, rtot = _LV[d]
    x3 = xf[..., off:off + sp].reshape(xf.shape[:-1] + (dp, dp, dp))
    return x3[..., 1:-1, 1:-1, 1:-1]


def _pool_slices(xf, d):
    x5 = _interior(xf, d)
    outs = []
    for a in (0, 1):
        for b in (0, 1):
            for c in (0, 1):
                outs.append(_embed(x5[..., a::2, b::2, c::2]))
    return jnp.stack(outs, axis=0)


def _conv_w(w):
    return jnp.transpose(w, (2, 3, 4, 0, 1)).reshape(27, w.shape[0], w.shape[1])


def _tconv_w(tw):
    twf = tw[:, :, ::-1, ::-1, ::-1]
    return jnp.transpose(twf, (2, 3, 4, 0, 1)).reshape(8, tw.shape[0], tw.shape[1])


def _col(v):
    return v.reshape(-1, 1)


def _row(v):
    return v.reshape(1, -1)


def _prep_body(lpat, li, v0, v1, v2, lts_out, inv_out):
    lp = jax.nn.sigmoid(lpat[...])
    n = jnp.sqrt(jnp.sum(lp * lp))
    lts_out[...] = jnp.dot(lp / n, li[...], preferred_element_type=jnp.float32)
    mx = [jnp.max(v[...]).reshape(1, 1) for v in (v0, v1, v2)]
    inv_out[...] = 1.0 / jnp.concatenate(mx, axis=0)


def _prep_call(lightpattern, light_info, vals):
    li = light_info.reshape(_H, _D * _D)
    v3 = [v.reshape(-1, 128) for v in vals]
    return pl.pallas_call(
        _prep_body,
        out_shape=[
            jax.ShapeDtypeStruct((_P, _D * _D), jnp.float32),
            jax.ShapeDtypeStruct((3, 1), jnp.float32),
        ],
    )(lightpattern, li, *v3)


def _sc_spmm_body(W, t0, t1, t2, g0, g1, g2, s0, s1, s2, v0, v1, v2, z,
                  o0, o1, o2, sh0, sh1, sh2, gi_v, si_v, va_v, gid, sid_,
                  gbuf, pbuf, sem):
    cid = lax.axis_index("c")
    sid = lax.axis_index("s")
    wid = cid * 16 + sid
    tables = (t0, t1, t2)
    gidx = (g0, g1, g2)
    sidx = (s0, s1, s2)
    vals = (v0, v1, v2)
    outs = (o0, o1, o2)
    shared = (sh0, sh1, sh2)
    stripe = (_N // 16) * W

    for m in range(3):
        pltpu.sync_copy(z, shared[m].at[pl.ds(sid * stripe, stripe)])
    plsc.subcore_barrier()

    jrows = (_NNZ // 32) // 128

    for m in range(3):
        row0 = wid * jrows
        pltpu.sync_copy(gidx[m].at[pl.ds(row0, jrows)], gi_v)
        pltpu.sync_copy(sidx[m].at[pl.ds(row0, jrows)], si_v)
        pltpu.sync_copy(vals[m].at[pl.ds(row0, jrows)], va_v)

        def body(j, _, m=m):
            for q in range(8):
                sl = pl.ds(q * 16, 16)
                g16 = gi_v[j, sl] * W
                s16 = si_v[j, sl] * W
                for c in range(W):
                    gid[c, sl] = g16 + c
                    sid_[c, sl] = s16 + c
            cps = [pltpu.async_copy(tables[m].at[gid.at[c]], gbuf.at[c], sem)
                   for c in range(W)]
            for cp in cps:
                cp.wait()
            for c in range(W):
                for q in range(8):
                    sl = pl.ds(q * 16, 16)
                    pbuf[c, sl] = gbuf[c, sl] * va_v[j, sl]
            for c in range(W):
                pltpu.sync_copy(pbuf.at[c], shared[m].at[sid_.at[c]],
                                add=True)
            return 0

        lax.fori_loop(0, jrows, body, 0)

    plsc.subcore_barrier()
    for m in range(3):
        pltpu.sync_copy(shared[m].at[pl.ds(sid * stripe, stripe)],
                        outs[m].at[pl.ds(cid * _N * W + sid * stripe,
                                         stripe)])


def _sc_spmm_call(tables, gidx, sidx, vals, W):
    mesh = plsc.VectorSubcoreMesh(core_axis_name="c", subcore_axis_name="s")
    nr = _NNZ // 128
    tf = [t.reshape(_N * W) for t in tables]
    gi = [g.reshape(nr, 128) for g in gidx]
    si = [s.reshape(nr, 128) for s in sidx]
    va = [v.reshape(nr, 128) for v in vals]
    z = jnp.zeros(((_N // 16) * W,), jnp.float32)
    jrows = (_NNZ // 32) // 128
    fn = pl.kernel(
        functools.partial(_sc_spmm_body, W),
        out_type=[jax.ShapeDtypeStruct((2 * _N * W,), jnp.float32)] * 3,
        mesh=mesh,
        scratch_types=[
            pltpu.VMEM_SHARED((_N * W,), jnp.float32),
            pltpu.VMEM_SHARED((_N * W,), jnp.float32),
            pltpu.VMEM_SHARED((_N * W,), jnp.float32),
            pltpu.VMEM((jrows, 128), jnp.int32),
            pltpu.VMEM((jrows, 128), jnp.int32),
            pltpu.VMEM((jrows, 128), jnp.float32),
            pltpu.VMEM((W, 128), jnp.int32),
            pltpu.VMEM((W, 128), jnp.int32),
            pltpu.VMEM((W, 128), jnp.float32),
            pltpu.VMEM((W, 128), jnp.float32),
            pltpu.SemaphoreType.DMA,
        ],
    )
    outs = fn(*tf, *gi, *si, *va, z)
    return [o.reshape(2 * _N, W) for o in outs]


def _mlp_body(xp, c0, c1, invr, w0a, w0b, b0, g0, e0, w1, b1, g1, e1,
              w2, b2, g2, e2, w3, b3, g3, e3, out):
    def bn_lrelu(h, g, e):
        m = jnp.mean(h, axis=0, keepdims=True)
        ym = h - m
        v = jnp.mean(ym * ym, axis=0, keepdims=True)
        t = g[...] * ym * lax.rsqrt(v + 1e-5) + e[...]
        return jnp.where(t >= 0, t, 0.01 * t)

    c = (c0[...] + c1[...]) * invr[...]
    h = (jnp.dot(xp[...], w0a[...], preferred_element_type=jnp.float32)
         + jnp.dot(c, w0b[...], preferred_element_type=jnp.float32) + b0[...])
    h = bn_lrelu(h, g0, e0)
    for w, b, g, e in ((w1, b1, g1, e1), (w2, b2, g2, e2), (w3, b3, g3, e3)):
        h = jnp.dot(h, w[...], preferred_element_type=jnp.float32) + b[...]
        h = bn_lrelu(h, g, e)
    out[...] = h


def _mlp_call(xpart, cexp0, cexp1, invrows, p):
    args = [xpart, cexp0, cexp1, invrows,
            p['mlp_w0'][:_P], p['mlp_w0'][_P:], _row(p['mlp_b0']),
            _row(p['mlp_g0']), _row(p['mlp_e0'])]
    for i in (1, 2, 3):
        args += [p['mlp_w%d' % i], _row(p['mlp_b%d' % i]),
                 _row(p['mlp_g%d' % i]), _row(p['mlp_e%d' % i])]
    rows = 3 * _B * _D * _D
    return pl.pallas_call(
        _mlp_body,
        out_shape=jax.ShapeDtypeStruct((rows, _FEAT[4]), jnp.float32),
    )(*args)


def _conv_acc(x_list, w_list, d):
    dp, sp, off, rtot = _LV[d]
    acc = None
    for x, w in zip(x_list, w_list):
        ki = 0
        for dd in (-1, 0, 1):
            for dh in (-1, 0, 1):
                for dw in (-1, 0, 1):
                    rel = off + dd * dp * dp + dh * dp + dw
                    xs = x[:, rel:rel + sp]
                    t = jnp.dot(w[ki], xs, preferred_element_type=jnp.float32)
                    acc = t if acc is None else acc + t
                    ki += 1
    return acc


def _colsum(y):
    return jnp.sum(y, axis=1, keepdims=True)


def _bn_relu(ys, g, e, mask, cnt):
    s = ys[0] if len(ys) == 1 else ys[0] + ys[1]
    m = _colsum(s) / cnt
    ym = [(y - m) * mask for y in ys]
    sq = ym[0] * ym[0] if len(ym) == 1 else ym[0] * ym[0] + ym[1] * ym[1]
    v = _colsum(sq) / cnt
    sc = g * lax.rsqrt(v + 1e-5)
    return [jnp.maximum(t * sc + e, 0.0) * mask for t in ym]


def _pad_lanes(y, d):
    dp, sp, off, rtot = _LV[d]
    z = jnp.zeros((y.shape[0], off), jnp.float32)
    return jnp.concatenate([z, y, z], axis=1)


def _dcb(xs_by_b, w1, b1, g1, e1, w2, b2, g2, e2, mask, d):
    cnt = _B * (d ** 3)
    ys = [_pad_lanes(_conv_acc(xs, w1, d) + b1[...], d) * mask
          for xs in xs_by_b]
    ys = _bn_relu(ys, g1[...], e1[...], mask, cnt)
    ys = [_pad_lanes(_conv_acc([y], [w2[...]], d) + b2[...], d) * mask
          for y in ys]
    return _bn_relu(ys, g2[...], e2[...], mask, cnt)


def _dc_args(p, nm):
    return [_conv_w(p[nm + '_w1']), _col(p[nm + '_b1']),
            _col(p[nm + '_g1']), _col(p[nm + '_e1']),
            _conv_w(p[nm + '_w2']), _col(p[nm + '_b2']),
            _col(p[nm + '_g2']), _col(p[nm + '_e2'])]


def _enc0_body(d, vp0, vp1, inv, w1, b1, g1, e1, w2, b2, g2, e2, mask, out):
    xs_by_b = [[(vp0[b] + vp1[b]) * inv[...]] for b in range(_B)]
    ys = _dcb(xs_by_b, [w1], b1, g1, e1, w2, b2, g2, e2, mask[...], d)
    for b in range(_B):
        out[b] = ys[b]


def _enc0_call(vp0, vp1, inv, p):
    d = _D
    rtot = _LV[d][3]
    return pl.pallas_call(
        functools.partial(_enc0_body, d),
        out_shape=jax.ShapeDtypeStruct((_B, _FC, rtot), jnp.float32),
    )(vp0, vp1, inv, *_dc_args(p, 'inc'), _MASKS[d])


def _enc_body(d, p8, w1, b1, g1, e1, w2, b2, g2, e2, mask, out):
    xs_by_b = []
    for b in range(_B):
        x = functools.reduce(jnp.maximum, [p8[k, b] for k in range(8)])
        xs_by_b.append([x])
    ys = _dcb(xs_by_b, [w1], b1, g1, e1, w2, b2, g2, e2, mask[...], d)
    for b in range(_B):
        out[b] = ys[b]


def _enc_call(p8, nm, cout, d, p):
    rtot = _LV[d][3]
    return pl.pallas_call(
        functools.partial(_enc_body, d),
        out_shape=jax.ShapeDtypeStruct((_B, cout, rtot), jnp.float32),
    )(p8, *_dc_args(p, nm), _MASKS[d])


def _tconv_body(xd, tw, tb, out):
    for k in range(8):
        for b in range(_B):
            out[k, b] = (jnp.dot(tw[k], xd[b],
                                 preferred_element_type=jnp.float32)
                         + tb[...])


def _tconv_call(xf, nm, d_in, p):
    xd = _interior(xf, d_in).reshape(_B, xf.shape[1], d_in ** 3)
    tw = _tconv_w(p[nm + '_tw'])
    cout = tw.shape[1]
    y8 = pl.pallas_call(
        _tconv_body,
        out_shape=jax.ShapeDtypeStruct((8, _B, cout, d_in ** 3), jnp.float32),
    )(xd, tw, _col(p[nm + '_tb']))
    n = d_in
    y = y8.reshape(2, 2, 2, _B, cout, n, n, n)
    y = jnp.transpose(y, (3, 4, 5, 0, 6, 1, 7, 2)).reshape(_B, cout, 2 * n,
                                                           2 * n, 2 * n)
    return _embed(y)


def _up_body(d, skip, up, w1s, w1u, b1, g1, e1, w2, b2, g2, e2, mask, out):
    xs_by_b = [[skip[b], up[b]] for b in range(_B)]
    ys = _dcb(xs_by_b, [w1s, w1u], b1, g1, e1, w2, b2, g2, e2, mask[...], d)
    for b in range(_B):
        out[b] = ys[b]


def _up_call(skip, up, nm, cout, d, p):
    rtot = _LV[d][3]
    a = _dc_args(p, nm)
    w1 = a[0]
    cs = skip.shape[1]
    args = [skip, up, w1[:, :, :cs], w1[:, :, cs:]] + a[1:] + [_MASKS[d]]
    return pl.pallas_call(
        functools.partial(_up_body, d),
        out_shape=jax.ShapeDtypeStruct((_B, cout, rtot), jnp.float32),
    )(*args)


def _u4_body(d, skip, up, w1s, w1u, b1, g1, e1, w2, b2, g2, e2, mask,
             ow, ob, out):
    xs_by_b = [[skip[b], up[b]] for b in range(_B)]
    ys = _dcb(xs_by_b, [w1s, w1u], b1, g1, e1, w2, b2, g2, e2, mask[...], d)
    for b in range(_B):
        out[b] = (jnp.dot(ow[...], ys[b], preferred_element_type=jnp.float32)
                  + ob[...])


def _u4_call(skip, up, p):
    d = _D
    rtot = _LV[d][3]
    a = _dc_args(p, 'u4')
    w1 = a[0]
    cs = skip.shape[1]
    ow = p['out_w'].reshape(1, _FC)
    ob = p['out_b'].reshape(1, 1)
    args = [skip, up, w1[:, :, :cs], w1[:, :, cs:]] + a[1:] + [_MASKS[d], ow, ob]
    return pl.pallas_call(
        functools.partial(_u4_body, d),
        out_shape=jax.ShapeDtypeStruct((_B, 1, rtot), jnp.float32),
    )(*args)


def kernel(measurement, lightpattern, light_info, rows0, cols0, vals0,
           rows1, cols1, vals1, rows2, cols2, vals2, params):
    p = params
    rows = [rows0, rows1, rows2]
    cols = [cols0, cols1, cols2]
    vals = [vals0, vals1, vals2]

    lts, inv = _prep_call(lightpattern, light_info, vals)
    ltf = jnp.broadcast_to(lts.reshape(_P, _D, 1, _D), (_P, _D, _D, _D))
    ltf = jnp.transpose(ltf, (1, 2, 3, 0)).reshape(_N, _P)

    parts = _sc_spmm_call([ltf] * 3, cols, rows, vals, _P)

    m5 = measurement.reshape(_B, _P, _C, _D, _D)
    xpart = jnp.stack([jnp.transpose(m5[:, :, i], (0, 2, 3, 1))
                       for i in range(_C)], 0).reshape(-1, _P)
    cexp = []
    for half in range(2):
        c3 = jnp.stack([parts[i][half * _N:(half + 1) * _N].reshape(
            _D * _D, _D * _P) for i in range(_C)], 0)
        cexp.append(jnp.broadcast_to(c3[:, None], (_C, _B, _D * _D, _D * _P))
                    .reshape(-1, _D * _P))
    invrows = jnp.repeat(inv, _B * _D * _D, axis=0)
    y = _mlp_call(xpart, cexp[0], cexp[1], invrows, p)

    yb = jnp.transpose(y.reshape(_C, _B, _D, _D, _D), (0, 2, 3, 4, 1))
    yb = [yb[i].reshape(_N, _B) for i in range(_C)]
    tparts = _sc_spmm_call(yb, rows, cols, vals, _B)

    vp = []
    for half in range(2):
        v = jnp.stack([jnp.transpose(tparts[i][half * _N:(half + 1) * _N]
                                     .reshape(_D, _D, _D, _B), (3, 0, 1, 2))
                       for i in range(_C)], 1)
        vp.append(_embed(v))
    xx1 = _enc0_call(vp[0], vp[1], inv, p)
    xx2 = _enc_call(_pool_slices(xx1, 32), 'd1', _FC * 2, 16, p)
    xx3 = _enc_call(_pool_slices(xx2, 16), 'd2', _FC * 4, 8, p)
    xx4 = _enc_call(_pool_slices(xx3, 8), 'd3', _FC * 8, 4, p)
    u = _up_call(xx3, _tconv_call(xx4, 'u2', 4, p), 'u2', _FC * 4, 8, p)
    u = _up_call(xx2, _tconv_call(u, 'u3', 8, p), 'u3', _FC * 2, 16, p)
    fin = _u4_call(xx1, _tconv_call(u, 'u4', 16, p), p)
    return _interior(fin, _D).reshape(_B, 1, _D, _D, _D)
